# baseline (speedup 1.0000x reference)
import functools

import jax
import jax.numpy as jnp
from jax import lax
from jax.experimental import pallas as pl
from jax.experimental.pallas import tpu as pltpu
from jax.experimental.pallas import tpu_sc as plsc

_B, _L, _V, _H, _O = 1024, 50, 100000, 64, 10
_NW = 32

_HB = _B // 2
_ROWS = _L * _HB
_BPW = _ROWS // _NW
_CHUNKS = 10
_CW = _BPW // _CHUNKS

_BBLK = 256
_GRID = _HB // _BBLK


def _sc_gather(table2, idx3):
    mesh = plsc.VectorSubcoreMesh(core_axis_name="c", subcore_axis_name="s")

    @functools.partial(
        pl.kernel,
        mesh=mesh,
        out_type=jax.ShapeDtypeStruct((_ROWS, 2 * _H), jnp.float32),
        scratch_types=[
            pltpu.VMEM((_CHUNKS, _CW), jnp.int32),
            pltpu.VMEM((_CW, 2 * _H), jnp.float32),
            pltpu.VMEM((_CW, 2 * _H), jnp.float32),
            pltpu.SemaphoreType.DMA,
            pltpu.SemaphoreType.DMA,
            pltpu.SemaphoreType.DMA,
            pltpu.SemaphoreType.DMA,
        ],
    )
    def k(table_hbm, idx_hbm, out_hbm, idx_v, buf0, buf1, gs0, gs1, os0, os1):
        wid = lax.axis_index("s") * 2 + lax.axis_index("c")
        base = wid * _BPW
        pltpu.sync_copy(idx_hbm.at[wid], idx_v)
        bufs = (buf0, buf1)
        gsems = (gs0, gs1)
        osems = (os0, os1)
        puts = [None, None]
        for j in range(_CHUNKS):
            b = j % 2
            if puts[b] is not None:
                puts[b].wait()
            g = pltpu.async_copy(table_hbm.at[idx_v.at[j]], bufs[b], gsems[b])
            g.wait()
            puts[b] = pltpu.async_copy(
                bufs[b], out_hbm.at[pl.ds(base + j * _CW, _CW)], osems[b]
            )
        puts[0].wait()
        puts[1].wait()

    return k(table2, idx3)


def _tc_body(emb_ref, par_ref, lens_ref, wxf_ref, bf_ref,
             wxb_ref, bb_ref, wo_ref, bo_ref,
             out_ref, hc_ref):
    lens = lens_ref[...]
    wxf = wxf_ref[...]
    bf = bf_ref[...]
    wxb = wxb_ref[...]
    bb = bb_ref[...]

    def xt_at(t):
        row = emb_ref[t]
        p = par_ref[t] != 0
        return jnp.where(p, row[:, _H:2 * _H], row[:, 0:_H])

    def cell(xt, h, c, wxh, b):
        xh = jnp.concatenate([xt, h], axis=1)
        g = jnp.dot(xh, wxh, preferred_element_type=jnp.float32) + b
        gi = jax.nn.sigmoid(g[:, 0:_H])
        gf = jax.nn.sigmoid(g[:, _H:2 * _H])
        gg = jnp.tanh(g[:, 2 * _H:3 * _H])
        go = jax.nn.sigmoid(g[:, 3 * _H:4 * _H])
        c_new = gf * c + gi * gg
        h_new = go * jnp.tanh(c_new)
        return h_new, c_new

    zeros = jnp.zeros((_BBLK, _H), jnp.float32)

    def step(s, carry):
        hf, cf, hb, cb = carry
        tb = _L - 1 - s
        hf_new, cf_new = cell(xt_at(s), hf, cf, wxf, bf)
        hb_new, cb_new = cell(xt_at(tb), hb, cb, wxb, bb)
        mf = lens > s
        mb = lens > tb
        hc_ref[s, :, 0:_H] = jnp.where(mf, hf_new, 0.0)
        hc_ref[tb, :, _H:2 * _H] = jnp.where(mb, hb_new, 0.0)
        return (jnp.where(mf, hf_new, hf), jnp.where(mf, cf_new, cf),
                jnp.where(mb, hb_new, hb), jnp.where(mb, cb_new, cb))

    lax.fori_loop(0, _L, step, (zeros, zeros, zeros, zeros))

    hcat = hc_ref[...].reshape(_L * _BBLK, 2 * _H)
    out_t = lax.dot_general(
        wo_ref[...], hcat, (((1,), (1,)), ((), ())),
        preferred_element_type=jnp.float32,
    ) + bo_ref[...]
    out_ref[...] = out_t.reshape(_O, _L, _BBLK)


def _tc_bilstm(emb_tm, par_tm, lens2, wxf, bf, wxb, bb, wo, bo):
    full = lambda shape: pl.BlockSpec(shape, lambda i: (0,) * len(shape))
    return pl.pallas_call(
        _tc_body,
        grid=(_GRID,),
        in_specs=[
            pl.BlockSpec((_L, _BBLK, 2 * _H), lambda i: (0, i, 0)),
            pl.BlockSpec((_L, _BBLK, 1), lambda i: (0, i, 0)),
            pl.BlockSpec((_BBLK, 1), lambda i: (i, 0)),
            full((2 * _H, 4 * _H)), full((1, 4 * _H)),
            full((2 * _H, 4 * _H)), full((1, 4 * _H)),
            full((_O, 2 * _H)), full((_O, 1)),
        ],
        out_specs=pl.BlockSpec((_O, _L, _BBLK), lambda i: (0, 0, i)),
        out_shape=jax.ShapeDtypeStruct((_O, _L, _HB), jnp.float32),
        scratch_shapes=[
            pltpu.VMEM((_L, _BBLK, 2 * _H), jnp.float32),
        ],
        compiler_params=pltpu.CompilerParams(
            dimension_semantics=("parallel",),
        ),
    )(emb_tm, par_tm, lens2, wxf, bf, wxb, bb, wo, bo)


def kernel(x, batch_seq_len, table, W_ih_f, W_hh_f, b_ih_f, b_hh_f,
           W_ih_b, W_hh_b, b_ih_b, b_hh_b, W_out, b_out):
    table2 = table.reshape(_V // 2, 2 * _H)
    lens = batch_seq_len.astype(jnp.int32)
    wxf = jnp.concatenate([W_ih_f.T, W_hh_f.T], axis=0)
    bf = (b_ih_f + b_hh_f)[None, :]
    wxb = jnp.concatenate([W_ih_b.T, W_hh_b.T], axis=0)
    bb = (b_ih_b + b_hh_b)[None, :]
    bo = b_out[:, None]

    outs = []
    for h in range(2):
        xh = x[h * _HB:(h + 1) * _HB]
        xt_flat = xh.T.reshape(-1)
        idx3 = (xt_flat >> 1).reshape(_NW, _CHUNKS, _CW)
        par_tm = (xt_flat & 1).astype(jnp.int8).reshape(_L, _HB, 1)
        emb_tm = _sc_gather(table2, idx3).reshape(_L, _HB, 2 * _H)
        lens2 = lens[h * _HB:(h + 1) * _HB, None]
        out_olb = _tc_bilstm(emb_tm, par_tm, lens2, wxf, bf, wxb, bb,
                             W_out, bo)
        outs.append(jnp.transpose(out_olb, (2, 1, 0)))
    return jnp.concatenate(outs, axis=0)


# speedup vs baseline: 1.2482x; 1.2482x over previous
import functools

import jax
import jax.numpy as jnp
from jax import lax
from jax.experimental import pallas as pl
from jax.experimental.pallas import tpu as pltpu
from jax.experimental.pallas import tpu_sc as plsc

_B, _L, _V, _H, _O = 1024, 50, 100000, 64, 10
_NW = 32

_HB = _B // 2
_ROWS = _L * _HB
_BPW = _ROWS // _NW
_CHUNKS = 10
_CW = _BPW // _CHUNKS

_BBLK = 512
_GRID = _HB // _BBLK


def _sc_gather(table2, idx3):
    mesh = plsc.VectorSubcoreMesh(core_axis_name="c", subcore_axis_name="s")

    @functools.partial(
        pl.kernel,
        mesh=mesh,
        out_type=jax.ShapeDtypeStruct((_ROWS, 2 * _H), jnp.float32),
        scratch_types=[
            pltpu.VMEM((_CHUNKS, _CW), jnp.int32),
            pltpu.VMEM((_CW, 2 * _H), jnp.float32),
            pltpu.VMEM((_CW, 2 * _H), jnp.float32),
            pltpu.SemaphoreType.DMA,
            pltpu.SemaphoreType.DMA,
            pltpu.SemaphoreType.DMA,
            pltpu.SemaphoreType.DMA,
        ],
    )
    def k(table_hbm, idx_hbm, out_hbm, idx_v, buf0, buf1, gs0, gs1, os0, os1):
        wid = lax.axis_index("s") * 2 + lax.axis_index("c")
        base = wid * _BPW
        pltpu.sync_copy(idx_hbm.at[wid], idx_v)
        bufs = (buf0, buf1)
        gsems = (gs0, gs1)
        osems = (os0, os1)
        puts = [None, None]
        for j in range(_CHUNKS):
            b = j % 2
            if puts[b] is not None:
                puts[b].wait()
            g = pltpu.async_copy(table_hbm.at[idx_v.at[j]], bufs[b], gsems[b])
            g.wait()
            puts[b] = pltpu.async_copy(
                bufs[b], out_hbm.at[pl.ds(base + j * _CW, _CW)], osems[b]
            )
        puts[0].wait()
        puts[1].wait()

    return k(table2, idx3)


def _tc_body(emb_ref, par_ref, lens_ref, wp_ref, bp_ref, wo_ref, bo_ref,
             out_ref, hc_ref):
    lens = lens_ref[...]
    wp = wp_ref[...]
    bp = bp_ref[...]
    lane = lax.broadcasted_iota(jnp.int32, (_BBLK, 2 * _H), 1)
    lo = lane < _H

    def xt_at(t):
        row = emb_ref[t]
        p = par_ref[t] != 0
        return jnp.where(p, row[:, _H:2 * _H], row[:, 0:_H])

    zeros = jnp.zeros((_BBLK, 2 * _H), jnp.float32)

    def step(s, carry):
        h, c = carry
        tb = _L - 1 - s
        z = jnp.concatenate([xt_at(s), xt_at(tb), h], axis=1)
        g = jnp.dot(z, wp, preferred_element_type=jnp.float32) + bp
        gi = jax.nn.sigmoid(g[:, 0:2 * _H])
        gf = jax.nn.sigmoid(g[:, 2 * _H:4 * _H])
        gg = jnp.tanh(g[:, 4 * _H:6 * _H])
        go = jax.nn.sigmoid(g[:, 6 * _H:8 * _H])
        c_new = gf * c + gi * gg
        h_new = go * jnp.tanh(c_new)
        m = lens > jnp.where(lo, s, tb)
        hm = jnp.where(m, h_new, 0.0)
        hc_ref[s, :, 0:_H] = hm[:, 0:_H]
        hc_ref[tb, :, _H:2 * _H] = hm[:, _H:2 * _H]
        return jnp.where(m, h_new, h), jnp.where(m, c_new, c)

    lax.fori_loop(0, _L, step, (zeros, zeros))

    hcat = hc_ref[...].reshape(_L * _BBLK, 2 * _H)
    out_t = lax.dot_general(
        wo_ref[...], hcat, (((1,), (1,)), ((), ())),
        preferred_element_type=jnp.float32,
    ) + bo_ref[...]
    out_ref[...] = out_t.reshape(_O, _L, _BBLK)


def _tc_bilstm(emb_tm, par_tm, lens2, wp, bp, wo, bo):
    full = lambda shape: pl.BlockSpec(shape, lambda i: (0,) * len(shape))
    return pl.pallas_call(
        _tc_body,
        grid=(_GRID,),
        in_specs=[
            pl.BlockSpec((_L, _BBLK, 2 * _H), lambda i: (0, i, 0)),
            pl.BlockSpec((_L, _BBLK, 1), lambda i: (0, i, 0)),
            pl.BlockSpec((_BBLK, 1), lambda i: (i, 0)),
            full((4 * _H, 8 * _H)), full((1, 8 * _H)),
            full((_O, 2 * _H)), full((_O, 1)),
        ],
        out_specs=pl.BlockSpec((_O, _L, _BBLK), lambda i: (0, 0, i)),
        out_shape=jax.ShapeDtypeStruct((_O, _L, _HB), jnp.float32),
        scratch_shapes=[
            pltpu.VMEM((_L, _BBLK, 2 * _H), jnp.float32),
        ],
        compiler_params=pltpu.CompilerParams(
            dimension_semantics=("parallel",),
        ),
    )(emb_tm, par_tm, lens2, wp, bp, wo, bo)


def kernel(x, batch_seq_len, table, W_ih_f, W_hh_f, b_ih_f, b_hh_f,
           W_ih_b, W_hh_b, b_ih_b, b_hh_b, W_out, b_out):
    table2 = table.reshape(_V // 2, 2 * _H)
    lens = batch_seq_len.astype(jnp.int32)
    wxf = jnp.concatenate([W_ih_f.T, W_hh_f.T], axis=0)
    bf = (b_ih_f + b_hh_f)[None, :]
    wxb = jnp.concatenate([W_ih_b.T, W_hh_b.T], axis=0)
    bb = (b_ih_b + b_hh_b)[None, :]
    bo = b_out[:, None]

    wxf4 = wxf.reshape(2, _H, 4, _H)
    wxb4 = wxb.reshape(2, _H, 4, _H)
    wp = jnp.zeros((4, _H, 4, 2, _H), jnp.float32)
    wp = wp.at[0, :, :, 0, :].set(wxf4[0])
    wp = wp.at[1, :, :, 1, :].set(wxb4[0])
    wp = wp.at[2, :, :, 0, :].set(wxf4[1])
    wp = wp.at[3, :, :, 1, :].set(wxb4[1])
    wp = wp.reshape(4 * _H, 8 * _H)
    bp = jnp.stack([bf.reshape(4, _H), bb.reshape(4, _H)],
                   axis=1).reshape(1, 8 * _H)

    outs = []
    for h in range(2):
        xh = x[h * _HB:(h + 1) * _HB]
        xt_flat = xh.T.reshape(-1)
        idx3 = (xt_flat >> 1).reshape(_NW, _CHUNKS, _CW)
        par_tm = (xt_flat & 1).astype(jnp.int8).reshape(_L, _HB, 1)
        emb_tm = _sc_gather(table2, idx3).reshape(_L, _HB, 2 * _H)
        lens2 = lens[h * _HB:(h + 1) * _HB, None]
        out_olb = _tc_bilstm(emb_tm, par_tm, lens2, wp, bp,
                             W_out, bo)
        outs.append(jnp.transpose(out_olb, (2, 1, 0)))
    return jnp.concatenate(outs, axis=0)


# speedup vs baseline: 1.2934x; 1.0362x over previous
import functools

import jax
import jax.numpy as jnp
from jax import lax
from jax.experimental import pallas as pl
from jax.experimental.pallas import tpu as pltpu
from jax.experimental.pallas import tpu_sc as plsc

_B, _L, _V, _H, _O = 1024, 50, 100000, 64, 10
_NW = 32

_HB = _B // 2
_ROWS = _L * _HB
_BPW = _ROWS // _NW
_CHUNKS = 10
_CW = _BPW // _CHUNKS

_BBLK = 512
_GRID = _HB // _BBLK


def _sc_gather(table2, idx3):
    mesh = plsc.VectorSubcoreMesh(core_axis_name="c", subcore_axis_name="s")

    @functools.partial(
        pl.kernel,
        mesh=mesh,
        out_type=jax.ShapeDtypeStruct((_ROWS, 2 * _H), jnp.float32),
        scratch_types=[
            pltpu.VMEM((_CHUNKS, _CW), jnp.int32),
            pltpu.VMEM((_CW, 2 * _H), jnp.float32),
            pltpu.VMEM((_CW, 2 * _H), jnp.float32),
            pltpu.SemaphoreType.DMA,
            pltpu.SemaphoreType.DMA,
            pltpu.SemaphoreType.DMA,
            pltpu.SemaphoreType.DMA,
        ],
    )
    def k(table_hbm, idx_hbm, out_hbm, idx_v, buf0, buf1, gs0, gs1, os0, os1):
        wid = lax.axis_index("s") * 2 + lax.axis_index("c")
        base = wid * _BPW
        pltpu.sync_copy(idx_hbm.at[wid], idx_v)
        bufs = (buf0, buf1)
        gsems = (gs0, gs1)
        osems = (os0, os1)
        puts = [None, None]
        for j in range(_CHUNKS):
            b = j % 2
            if puts[b] is not None:
                puts[b].wait()
            g = pltpu.async_copy(table_hbm.at[idx_v.at[j]], bufs[b], gsems[b])
            g.wait()
            puts[b] = pltpu.async_copy(
                bufs[b], out_hbm.at[pl.ds(base + j * _CW, _CW)], osems[b]
            )
        puts[0].wait()
        puts[1].wait()

    return k(table2, idx3)


def _tc_body(emb_ref, par_ref, lens_ref, wp_ref, bp_ref, wo_ref, bo_ref,
             out_ref, hc_ref):
    lens = lens_ref[...]
    wp = wp_ref[...]
    bp = bp_ref[...]
    lane = lax.broadcasted_iota(jnp.int32, (_BBLK, 2 * _H), 1)
    lo = lane < _H

    def xt_at(t):
        row = emb_ref[t]
        p = par_ref[t] != 0
        return jnp.where(p, row[:, _H:2 * _H], row[:, 0:_H])

    zeros = jnp.zeros((_BBLK, 2 * _H), jnp.float32)

    def step(i, carry):
        h, c = carry
        for k in range(2):
            s = 2 * i + k
            tb = _L - 1 - s
            xx = jnp.concatenate([xt_at(s), xt_at(tb)], axis=1)
            gx = jnp.dot(xx, wp[0:2 * _H], preferred_element_type=jnp.float32)
            g = gx + jnp.dot(h, wp[2 * _H:4 * _H],
                             preferred_element_type=jnp.float32) + bp
            gi = jax.nn.sigmoid(g[:, 0:2 * _H])
            gf = jax.nn.sigmoid(g[:, 2 * _H:4 * _H])
            gg = jnp.tanh(g[:, 4 * _H:6 * _H])
            go = jax.nn.sigmoid(g[:, 6 * _H:8 * _H])
            c_new = gf * c + gi * gg
            h_new = go * jnp.tanh(c_new)
            m = lens > jnp.where(lo, s, tb)
            hm = jnp.where(m, h_new, 0.0)
            hc_ref[s, :, 0:_H] = hm[:, 0:_H]
            hc_ref[tb, :, _H:2 * _H] = hm[:, _H:2 * _H]
            h = jnp.where(m, h_new, h)
            c = jnp.where(m, c_new, c)
        return h, c

    lax.fori_loop(0, _L // 2, step, (zeros, zeros))

    hcat = hc_ref[...].reshape(_L * _BBLK, 2 * _H)
    out_t = lax.dot_general(
        wo_ref[...], hcat, (((1,), (1,)), ((), ())),
        preferred_element_type=jnp.float32,
    ) + bo_ref[...]
    out_ref[...] = out_t.reshape(_O, _L, _BBLK)


def _tc_bilstm(emb_tm, par_tm, lens2, wp, bp, wo, bo):
    full = lambda shape: pl.BlockSpec(shape, lambda i: (0,) * len(shape))
    return pl.pallas_call(
        _tc_body,
        grid=(_GRID,),
        in_specs=[
            pl.BlockSpec((_L, _BBLK, 2 * _H), lambda i: (0, i, 0)),
            pl.BlockSpec((_L, _BBLK, 1), lambda i: (0, i, 0)),
            pl.BlockSpec((_BBLK, 1), lambda i: (i, 0)),
            full((4 * _H, 8 * _H)), full((1, 8 * _H)),
            full((_O, 2 * _H)), full((_O, 1)),
        ],
        out_specs=pl.BlockSpec((_O, _L, _BBLK), lambda i: (0, 0, i)),
        out_shape=jax.ShapeDtypeStruct((_O, _L, _HB), jnp.float32),
        scratch_shapes=[
            pltpu.VMEM((_L, _BBLK, 2 * _H), jnp.float32),
        ],
        compiler_params=pltpu.CompilerParams(
            dimension_semantics=("parallel",),
        ),
    )(emb_tm, par_tm, lens2, wp, bp, wo, bo)


def kernel(x, batch_seq_len, table, W_ih_f, W_hh_f, b_ih_f, b_hh_f,
           W_ih_b, W_hh_b, b_ih_b, b_hh_b, W_out, b_out):
    table2 = table.reshape(_V // 2, 2 * _H)
    lens = batch_seq_len.astype(jnp.int32)
    wxf = jnp.concatenate([W_ih_f.T, W_hh_f.T], axis=0)
    bf = (b_ih_f + b_hh_f)[None, :]
    wxb = jnp.concatenate([W_ih_b.T, W_hh_b.T], axis=0)
    bb = (b_ih_b + b_hh_b)[None, :]
    bo = b_out[:, None]

    wxf4 = wxf.reshape(2, _H, 4, _H)
    wxb4 = wxb.reshape(2, _H, 4, _H)
    wp = jnp.zeros((4, _H, 4, 2, _H), jnp.float32)
    wp = wp.at[0, :, :, 0, :].set(wxf4[0])
    wp = wp.at[1, :, :, 1, :].set(wxb4[0])
    wp = wp.at[2, :, :, 0, :].set(wxf4[1])
    wp = wp.at[3, :, :, 1, :].set(wxb4[1])
    wp = wp.reshape(4 * _H, 8 * _H)
    bp = jnp.stack([bf.reshape(4, _H), bb.reshape(4, _H)],
                   axis=1).reshape(1, 8 * _H)

    outs = []
    for h in range(2):
        xh = x[h * _HB:(h + 1) * _HB]
        xt_flat = xh.T.reshape(-1)
        idx3 = (xt_flat >> 1).reshape(_NW, _CHUNKS, _CW)
        par_tm = (xt_flat & 1).astype(jnp.int8).reshape(_L, _HB, 1)
        emb_tm = _sc_gather(table2, idx3).reshape(_L, _HB, 2 * _H)
        lens2 = lens[h * _HB:(h + 1) * _HB, None]
        out_olb = _tc_bilstm(emb_tm, par_tm, lens2, wp, bp,
                             W_out, bo)
        outs.append(jnp.transpose(out_olb, (2, 1, 0)))
    return jnp.concatenate(outs, axis=0)


# speedup vs baseline: 1.3430x; 1.0383x over previous
import functools

import jax
import jax.numpy as jnp
from jax import lax
from jax.experimental import pallas as pl
from jax.experimental.pallas import tpu as pltpu
from jax.experimental.pallas import tpu_sc as plsc

_B, _L, _V, _H, _O = 1024, 50, 100000, 64, 10
_NW = 32

_HB = _B // 2
_ROWS = _L * _HB
_BPW = _ROWS // _NW
_CHUNKS = 10
_CW = _BPW // _CHUNKS

_BBLK = 512
_GRID = _HB // _BBLK


def _sc_gather(table2, idx3):
    mesh = plsc.VectorSubcoreMesh(core_axis_name="c", subcore_axis_name="s")

    @functools.partial(
        pl.kernel,
        mesh=mesh,
        out_type=jax.ShapeDtypeStruct((_ROWS, 2 * _H), jnp.float32),
        scratch_types=[
            pltpu.VMEM((_CHUNKS, _CW), jnp.int32),
            pltpu.VMEM((_CW, 2 * _H), jnp.float32),
            pltpu.VMEM((_CW, 2 * _H), jnp.float32),
            pltpu.SemaphoreType.DMA,
            pltpu.SemaphoreType.DMA,
            pltpu.SemaphoreType.DMA,
            pltpu.SemaphoreType.DMA,
        ],
    )
    def k(table_hbm, idx_hbm, out_hbm, idx_v, buf0, buf1, gs0, gs1, os0, os1):
        wid = lax.axis_index("s") * 2 + lax.axis_index("c")
        base = wid * _BPW
        pltpu.sync_copy(idx_hbm.at[wid], idx_v)
        bufs = (buf0, buf1)
        gsems = (gs0, gs1)
        osems = (os0, os1)
        puts = [None, None]
        for j in range(_CHUNKS):
            b = j % 2
            if puts[b] is not None:
                puts[b].wait()
            g = pltpu.async_copy(table_hbm.at[idx_v.at[j]], bufs[b], gsems[b])
            g.wait()
            puts[b] = pltpu.async_copy(
                bufs[b], out_hbm.at[pl.ds(base + j * _CW, _CW)], osems[b]
            )
        puts[0].wait()
        puts[1].wait()

    return k(table2, idx3)


def _tc_body(emb_ref, par_ref, lens_ref, wp_ref, bp_ref, wo_ref, bo_ref,
             out_ref, hc_ref):
    lens = lens_ref[...]
    wp = wp_ref[...]
    bp = bp_ref[...]
    lane = lax.broadcasted_iota(jnp.int32, (_BBLK, 2 * _H), 1)
    lo = lane < _H

    def xt_at(t):
        row = emb_ref[t]
        p = par_ref[t] != 0
        return jnp.where(p, row[:, _H:2 * _H], row[:, 0:_H])

    zeros = jnp.zeros((_BBLK, 2 * _H), jnp.float32)

    def step(i, carry):
        h, c = carry
        for k in range(5):
            s = 5 * i + k
            tb = _L - 1 - s
            xx = jnp.concatenate([xt_at(s), xt_at(tb)], axis=1)
            gx = jnp.dot(xx, wp[0:2 * _H], preferred_element_type=jnp.float32)
            g = gx + jnp.dot(h, wp[2 * _H:4 * _H],
                             preferred_element_type=jnp.float32) + bp
            gi = jax.nn.sigmoid(g[:, 0:2 * _H])
            gf = jax.nn.sigmoid(g[:, 2 * _H:4 * _H])
            gg = jnp.tanh(g[:, 4 * _H:6 * _H])
            go = jax.nn.sigmoid(g[:, 6 * _H:8 * _H])
            c_new = gf * c + gi * gg
            h_new = go * jnp.tanh(c_new)
            m = lens > jnp.where(lo, s, tb)
            hm = jnp.where(m, h_new, 0.0)
            hc_ref[s, :, 0:_H] = hm[:, 0:_H]
            hc_ref[tb, :, _H:2 * _H] = hm[:, _H:2 * _H]
            h = jnp.where(m, h_new, h)
            c = jnp.where(m, c_new, c)
        return h, c

    lax.fori_loop(0, _L // 5, step, (zeros, zeros))

    hcat = hc_ref[...].reshape(_L * _BBLK, 2 * _H)
    out_t = lax.dot_general(
        wo_ref[...], hcat, (((1,), (1,)), ((), ())),
        preferred_element_type=jnp.float32,
    ) + bo_ref[...]
    out_ref[...] = out_t.reshape(_O, _L, _BBLK)


def _tc_bilstm(emb_tm, par_tm, lens2, wp, bp, wo, bo):
    full = lambda shape: pl.BlockSpec(shape, lambda i: (0,) * len(shape))
    return pl.pallas_call(
        _tc_body,
        grid=(_GRID,),
        in_specs=[
            pl.BlockSpec((_L, _BBLK, 2 * _H), lambda i: (0, i, 0)),
            pl.BlockSpec((_L, _BBLK, 1), lambda i: (0, i, 0)),
            pl.BlockSpec((_BBLK, 1), lambda i: (i, 0)),
            full((4 * _H, 8 * _H)), full((1, 8 * _H)),
            full((_O, 2 * _H)), full((_O, 1)),
        ],
        out_specs=pl.BlockSpec((_O, _L, _BBLK), lambda i: (0, 0, i)),
        out_shape=jax.ShapeDtypeStruct((_O, _L, _HB), jnp.float32),
        scratch_shapes=[
            pltpu.VMEM((_L, _BBLK, 2 * _H), jnp.float32),
        ],
        compiler_params=pltpu.CompilerParams(
            dimension_semantics=("parallel",),
        ),
    )(emb_tm, par_tm, lens2, wp, bp, wo, bo)


def kernel(x, batch_seq_len, table, W_ih_f, W_hh_f, b_ih_f, b_hh_f,
           W_ih_b, W_hh_b, b_ih_b, b_hh_b, W_out, b_out):
    table2 = table.reshape(_V // 2, 2 * _H)
    lens = batch_seq_len.astype(jnp.int32)
    wxf = jnp.concatenate([W_ih_f.T, W_hh_f.T], axis=0)
    bf = (b_ih_f + b_hh_f)[None, :]
    wxb = jnp.concatenate([W_ih_b.T, W_hh_b.T], axis=0)
    bb = (b_ih_b + b_hh_b)[None, :]
    bo = b_out[:, None]

    wxf4 = wxf.reshape(2, _H, 4, _H)
    wxb4 = wxb.reshape(2, _H, 4, _H)
    wp = jnp.zeros((4, _H, 4, 2, _H), jnp.float32)
    wp = wp.at[0, :, :, 0, :].set(wxf4[0])
    wp = wp.at[1, :, :, 1, :].set(wxb4[0])
    wp = wp.at[2, :, :, 0, :].set(wxf4[1])
    wp = wp.at[3, :, :, 1, :].set(wxb4[1])
    wp = wp.reshape(4 * _H, 8 * _H)
    bp = jnp.stack([bf.reshape(4, _H), bb.reshape(4, _H)],
                   axis=1).reshape(1, 8 * _H)

    outs = []
    for h in range(2):
        xh = x[h * _HB:(h + 1) * _HB]
        xt_flat = xh.T.reshape(-1)
        idx3 = (xt_flat >> 1).reshape(_NW, _CHUNKS, _CW)
        par_tm = (xt_flat & 1).astype(jnp.int8).reshape(_L, _HB, 1)
        emb_tm = _sc_gather(table2, idx3).reshape(_L, _HB, 2 * _H)
        lens2 = lens[h * _HB:(h + 1) * _HB, None]
        out_olb = _tc_bilstm(emb_tm, par_tm, lens2, wp, bp,
                             W_out, bo)
        outs.append(jnp.transpose(out_olb, (2, 1, 0)))
    return jnp.concatenate(outs, axis=0)


# speedup vs baseline: 1.3639x; 1.0156x over previous
import functools

import jax
import jax.numpy as jnp
from jax import lax
from jax.experimental import pallas as pl
from jax.experimental.pallas import tpu as pltpu
from jax.experimental.pallas import tpu_sc as plsc

_B, _L, _V, _H, _O = 1024, 50, 100000, 64, 10
_NW = 32

_HB = _B // 2
_ROWS = _L * _HB
_BPW = _ROWS // _NW
_CHUNKS = 10
_CW = _BPW // _CHUNKS

_BBLK = 512
_GRID = _HB // _BBLK


def _sc_gather(table2, idx3):
    mesh = plsc.VectorSubcoreMesh(core_axis_name="c", subcore_axis_name="s")

    @functools.partial(
        pl.kernel,
        mesh=mesh,
        out_type=jax.ShapeDtypeStruct((_ROWS, 2 * _H), jnp.float32),
        scratch_types=[
            pltpu.VMEM((_CHUNKS, _CW), jnp.int32),
            pltpu.VMEM((_CW, 2 * _H), jnp.float32),
            pltpu.VMEM((_CW, 2 * _H), jnp.float32),
            pltpu.SemaphoreType.DMA,
            pltpu.SemaphoreType.DMA,
            pltpu.SemaphoreType.DMA,
            pltpu.SemaphoreType.DMA,
        ],
    )
    def k(table_hbm, idx_hbm, out_hbm, idx_v, buf0, buf1, gs0, gs1, os0, os1):
        wid = lax.axis_index("s") * 2 + lax.axis_index("c")
        base = wid * _BPW
        pltpu.sync_copy(idx_hbm.at[wid], idx_v)
        bufs = (buf0, buf1)
        gsems = (gs0, gs1)
        osems = (os0, os1)
        puts = [None, None]
        for j in range(_CHUNKS):
            b = j % 2
            if puts[b] is not None:
                puts[b].wait()
            g = pltpu.async_copy(table_hbm.at[idx_v.at[j]], bufs[b], gsems[b])
            g.wait()
            puts[b] = pltpu.async_copy(
                bufs[b], out_hbm.at[pl.ds(base + j * _CW, _CW)], osems[b]
            )
        puts[0].wait()
        puts[1].wait()

    return k(table2, idx3)


def _tc_body(emb_ref, par_ref, lens_ref, wp_ref, bp_ref, wo_ref, bo_ref,
             out_ref, hc_ref):
    lens = lens_ref[...]
    wp = wp_ref[...]
    bp = bp_ref[...]
    lane = lax.broadcasted_iota(jnp.int32, (_BBLK, 2 * _H), 1)
    lo = lane < _H

    def xt_at(t):
        row = emb_ref[t]
        p = par_ref[t] != 0
        return jnp.where(p, row[:, _H:2 * _H], row[:, 0:_H])

    zeros = jnp.zeros((_BBLK, 2 * _H), jnp.float32)

    def step(i, carry):
        h, c = carry
        for k in range(10):
            s = 10 * i + k
            tb = _L - 1 - s
            xx = jnp.concatenate([xt_at(s), xt_at(tb)], axis=1)
            gx = jnp.dot(xx, wp[0:2 * _H], preferred_element_type=jnp.float32)
            g = gx + jnp.dot(h, wp[2 * _H:4 * _H],
                             preferred_element_type=jnp.float32) + bp
            gi = jax.nn.sigmoid(g[:, 0:2 * _H])
            gf = jax.nn.sigmoid(g[:, 2 * _H:4 * _H])
            gg = jnp.tanh(g[:, 4 * _H:6 * _H])
            go = jax.nn.sigmoid(g[:, 6 * _H:8 * _H])
            c_new = gf * c + gi * gg
            h_new = go * jnp.tanh(c_new)
            m = lens > jnp.where(lo, s, tb)
            hm = jnp.where(m, h_new, 0.0)
            hc_ref[s, :, 0:_H] = hm[:, 0:_H]
            hc_ref[tb, :, _H:2 * _H] = hm[:, _H:2 * _H]
            h = jnp.where(m, h_new, h)
            c = jnp.where(m, c_new, c)
        return h, c

    lax.fori_loop(0, _L // 10, step, (zeros, zeros))

    hcat = hc_ref[...].reshape(_L * _BBLK, 2 * _H)
    out_t = lax.dot_general(
        wo_ref[...], hcat, (((1,), (1,)), ((), ())),
        preferred_element_type=jnp.float32,
    ) + bo_ref[...]
    out_ref[...] = out_t.reshape(_O, _L, _BBLK)


def _tc_bilstm(emb_tm, par_tm, lens2, wp, bp, wo, bo):
    full = lambda shape: pl.BlockSpec(shape, lambda i: (0,) * len(shape))
    return pl.pallas_call(
        _tc_body,
        grid=(_GRID,),
        in_specs=[
            pl.BlockSpec((_L, _BBLK, 2 * _H), lambda i: (0, i, 0)),
            pl.BlockSpec((_L, _BBLK, 1), lambda i: (0, i, 0)),
            pl.BlockSpec((_BBLK, 1), lambda i: (i, 0)),
            full((4 * _H, 8 * _H)), full((1, 8 * _H)),
            full((_O, 2 * _H)), full((_O, 1)),
        ],
        out_specs=pl.BlockSpec((_O, _L, _BBLK), lambda i: (0, 0, i)),
        out_shape=jax.ShapeDtypeStruct((_O, _L, _HB), jnp.float32),
        scratch_shapes=[
            pltpu.VMEM((_L, _BBLK, 2 * _H), jnp.float32),
        ],
        compiler_params=pltpu.CompilerParams(
            dimension_semantics=("parallel",),
        ),
    )(emb_tm, par_tm, lens2, wp, bp, wo, bo)


def kernel(x, batch_seq_len, table, W_ih_f, W_hh_f, b_ih_f, b_hh_f,
           W_ih_b, W_hh_b, b_ih_b, b_hh_b, W_out, b_out):
    table2 = table.reshape(_V // 2, 2 * _H)
    lens = batch_seq_len.astype(jnp.int32)
    wxf = jnp.concatenate([W_ih_f.T, W_hh_f.T], axis=0)
    bf = (b_ih_f + b_hh_f)[None, :]
    wxb = jnp.concatenate([W_ih_b.T, W_hh_b.T], axis=0)
    bb = (b_ih_b + b_hh_b)[None, :]
    bo = b_out[:, None]

    wxf4 = wxf.reshape(2, _H, 4, _H)
    wxb4 = wxb.reshape(2, _H, 4, _H)
    wp = jnp.zeros((4, _H, 4, 2, _H), jnp.float32)
    wp = wp.at[0, :, :, 0, :].set(wxf4[0])
    wp = wp.at[1, :, :, 1, :].set(wxb4[0])
    wp = wp.at[2, :, :, 0, :].set(wxf4[1])
    wp = wp.at[3, :, :, 1, :].set(wxb4[1])
    wp = wp.reshape(4 * _H, 8 * _H)
    bp = jnp.stack([bf.reshape(4, _H), bb.reshape(4, _H)],
                   axis=1).reshape(1, 8 * _H)

    outs = []
    for h in range(2):
        xh = x[h * _HB:(h + 1) * _HB]
        xt_flat = xh.T.reshape(-1)
        idx3 = (xt_flat >> 1).reshape(_NW, _CHUNKS, _CW)
        par_tm = (xt_flat & 1).astype(jnp.int8).reshape(_L, _HB, 1)
        emb_tm = _sc_gather(table2, idx3).reshape(_L, _HB, 2 * _H)
        lens2 = lens[h * _HB:(h + 1) * _HB, None]
        out_olb = _tc_bilstm(emb_tm, par_tm, lens2, wp, bp,
                             W_out, bo)
        outs.append(jnp.transpose(out_olb, (2, 1, 0)))
    return jnp.concatenate(outs, axis=0)


# speedup vs baseline: 1.3814x; 1.0129x over previous
import functools

import jax
import jax.numpy as jnp
from jax import lax
from jax.experimental import pallas as pl
from jax.experimental.pallas import tpu as pltpu
from jax.experimental.pallas import tpu_sc as plsc

_B, _L, _V, _H, _O = 1024, 50, 100000, 64, 10
_NW = 32

_HB = _B // 2
_ROWS = _L * _HB
_BPW = _ROWS // _NW
_CHUNKS = 10
_CW = _BPW // _CHUNKS

_BBLK = 512
_GRID = _HB // _BBLK


def _sc_gather(table2, idx3):
    mesh = plsc.VectorSubcoreMesh(core_axis_name="c", subcore_axis_name="s")

    @functools.partial(
        pl.kernel,
        mesh=mesh,
        out_type=jax.ShapeDtypeStruct((_ROWS, 2 * _H), jnp.float32),
        scratch_types=[
            pltpu.VMEM((_CHUNKS, _CW), jnp.int32),
            pltpu.VMEM((_CW, 2 * _H), jnp.float32),
            pltpu.VMEM((_CW, 2 * _H), jnp.float32),
            pltpu.SemaphoreType.DMA,
            pltpu.SemaphoreType.DMA,
            pltpu.SemaphoreType.DMA,
            pltpu.SemaphoreType.DMA,
        ],
    )
    def k(table_hbm, idx_hbm, out_hbm, idx_v, buf0, buf1, gs0, gs1, os0, os1):
        wid = lax.axis_index("s") * 2 + lax.axis_index("c")
        base = wid * _BPW
        pltpu.sync_copy(idx_hbm.at[wid], idx_v)
        bufs = (buf0, buf1)
        gsems = (gs0, gs1)
        osems = (os0, os1)
        puts = [None, None]
        for j in range(_CHUNKS):
            b = j % 2
            if puts[b] is not None:
                puts[b].wait()
            g = pltpu.async_copy(table_hbm.at[idx_v.at[j]], bufs[b], gsems[b])
            g.wait()
            puts[b] = pltpu.async_copy(
                bufs[b], out_hbm.at[pl.ds(base + j * _CW, _CW)], osems[b]
            )
        puts[0].wait()
        puts[1].wait()

    return k(table2, idx3)


def _tc_body(emb_ref, par_ref, lens_ref, wp_ref, bp_ref, wo_ref, bo_ref,
             out_ref, hc_ref):
    lens = lens_ref[...]
    wp = wp_ref[...]
    bp = bp_ref[...]
    lane = lax.broadcasted_iota(jnp.int32, (_BBLK, 2 * _H), 1)
    lo = lane < _H

    def xt_at(t):
        row = emb_ref[t]
        p = par_ref[t] != 0
        return jnp.where(p, row[:, _H:2 * _H], row[:, 0:_H])

    zeros = jnp.zeros((_BBLK, 2 * _H), jnp.float32)

    def step(i, carry):
        h, c = carry
        for k in range(25):
            s = 25 * i + k
            tb = _L - 1 - s
            xx = jnp.concatenate([xt_at(s), xt_at(tb)], axis=1)
            gx = jnp.dot(xx, wp[0:2 * _H], preferred_element_type=jnp.float32)
            g = gx + jnp.dot(h, wp[2 * _H:4 * _H],
                             preferred_element_type=jnp.float32) + bp
            gi = jax.nn.sigmoid(g[:, 0:2 * _H])
            gf = jax.nn.sigmoid(g[:, 2 * _H:4 * _H])
            gg = jnp.tanh(g[:, 4 * _H:6 * _H])
            go = jax.nn.sigmoid(g[:, 6 * _H:8 * _H])
            c_new = gf * c + gi * gg
            h_new = go * jnp.tanh(c_new)
            m = lens > jnp.where(lo, s, tb)
            hm = jnp.where(m, h_new, 0.0)
            hc_ref[s, :, 0:_H] = hm[:, 0:_H]
            hc_ref[tb, :, _H:2 * _H] = hm[:, _H:2 * _H]
            h = jnp.where(m, h_new, h)
            c = jnp.where(m, c_new, c)
        return h, c

    lax.fori_loop(0, _L // 25, step, (zeros, zeros))

    hcat = hc_ref[...].reshape(_L * _BBLK, 2 * _H)
    out_t = lax.dot_general(
        wo_ref[...], hcat, (((1,), (1,)), ((), ())),
        preferred_element_type=jnp.float32,
    ) + bo_ref[...]
    out_ref[...] = out_t.reshape(_O, _L, _BBLK)


def _tc_bilstm(emb_tm, par_tm, lens2, wp, bp, wo, bo):
    full = lambda shape: pl.BlockSpec(shape, lambda i: (0,) * len(shape))
    return pl.pallas_call(
        _tc_body,
        grid=(_GRID,),
        in_specs=[
            pl.BlockSpec((_L, _BBLK, 2 * _H), lambda i: (0, i, 0)),
            pl.BlockSpec((_L, _BBLK, 1), lambda i: (0, i, 0)),
            pl.BlockSpec((_BBLK, 1), lambda i: (i, 0)),
            full((4 * _H, 8 * _H)), full((1, 8 * _H)),
            full((_O, 2 * _H)), full((_O, 1)),
        ],
        out_specs=pl.BlockSpec((_O, _L, _BBLK), lambda i: (0, 0, i)),
        out_shape=jax.ShapeDtypeStruct((_O, _L, _HB), jnp.float32),
        scratch_shapes=[
            pltpu.VMEM((_L, _BBLK, 2 * _H), jnp.float32),
        ],
        compiler_params=pltpu.CompilerParams(
            dimension_semantics=("parallel",),
        ),
    )(emb_tm, par_tm, lens2, wp, bp, wo, bo)


def kernel(x, batch_seq_len, table, W_ih_f, W_hh_f, b_ih_f, b_hh_f,
           W_ih_b, W_hh_b, b_ih_b, b_hh_b, W_out, b_out):
    table2 = table.reshape(_V // 2, 2 * _H)
    lens = batch_seq_len.astype(jnp.int32)
    wxf = jnp.concatenate([W_ih_f.T, W_hh_f.T], axis=0)
    bf = (b_ih_f + b_hh_f)[None, :]
    wxb = jnp.concatenate([W_ih_b.T, W_hh_b.T], axis=0)
    bb = (b_ih_b + b_hh_b)[None, :]
    bo = b_out[:, None]

    wxf4 = wxf.reshape(2, _H, 4, _H)
    wxb4 = wxb.reshape(2, _H, 4, _H)
    wp = jnp.zeros((4, _H, 4, 2, _H), jnp.float32)
    wp = wp.at[0, :, :, 0, :].set(wxf4[0])
    wp = wp.at[1, :, :, 1, :].set(wxb4[0])
    wp = wp.at[2, :, :, 0, :].set(wxf4[1])
    wp = wp.at[3, :, :, 1, :].set(wxb4[1])
    wp = wp.reshape(4 * _H, 8 * _H)
    bp = jnp.stack([bf.reshape(4, _H), bb.reshape(4, _H)],
                   axis=1).reshape(1, 8 * _H)

    outs = []
    for h in range(2):
        xh = x[h * _HB:(h + 1) * _HB]
        xt_flat = xh.T.reshape(-1)
        idx3 = (xt_flat >> 1).reshape(_NW, _CHUNKS, _CW)
        par_tm = (xt_flat & 1).astype(jnp.int8).reshape(_L, _HB, 1)
        emb_tm = _sc_gather(table2, idx3).reshape(_L, _HB, 2 * _H)
        lens2 = lens[h * _HB:(h + 1) * _HB, None]
        out_olb = _tc_bilstm(emb_tm, par_tm, lens2, wp, bp,
                             W_out, bo)
        outs.append(jnp.transpose(out_olb, (2, 1, 0)))
    return jnp.concatenate(outs, axis=0)
